# baseline (speedup 1.0000x reference)
import functools

import jax
from jax import lax
import jax.numpy as jnp
from jax.experimental import pallas as pl
from jax.experimental.pallas import tpu as pltpu
from jax.experimental.pallas import tpu_sc as plsc

NC, NS = 2, 16
NW = NC * NS
CHUNK = 32
NBUF = 4


def kernel(input_ids, table):
    batch, seq = input_ids.shape
    n = batch * seq
    _, d = table.shape
    b_per_w = n // NW
    nchunk = b_per_w // CHUNK
    idx = input_ids.reshape(n).astype(jnp.int32)

    mesh = plsc.VectorSubcoreMesh(core_axis_name="c", subcore_axis_name="s")

    @functools.partial(
        pl.kernel,
        out_type=jax.ShapeDtypeStruct((n, d), table.dtype),
        mesh=mesh,
        scratch_types=[
            pltpu.VMEM((b_per_w,), jnp.int32),
            pltpu.VMEM((NBUF, CHUNK, d), jnp.float32),
        ] + [pltpu.SemaphoreType.DMA] * (2 * NBUF),
    )
    def gather_kernel(tab_hbm, idx_hbm, out_hbm, idx_v, rows_v, *sems):
        gsems = sems[:NBUF]
        osems = sems[NBUF:]
        wid = lax.axis_index("s") * NC + lax.axis_index("c")
        base = wid * b_per_w
        pltpu.sync_copy(idx_hbm.at[pl.ds(base, b_per_w)], idx_v)

        def gather_cp(g, b):
            return pltpu.make_async_copy(
                tab_hbm.at[idx_v.at[pl.ds(g * CHUNK, CHUNK)]],
                rows_v.at[b], gsems[b])

        def out_cp(g, b):
            return pltpu.make_async_copy(
                rows_v.at[b], out_hbm.at[pl.ds(base + g * CHUNK, CHUNK)],
                osems[b])

        for b in range(NBUF):
            gather_cp(b, b).start()

        @pl.loop(0, nchunk, step=NBUF)
        def _(c):
            for b in range(NBUF):
                g = c + b
                gather_cp(g, b).wait()
                out_cp(g, b).start()
            for b in range(NBUF):
                g = c + b

                @pl.when(g + NBUF < nchunk)
                def _():
                    out_cp(g, b).wait()
                    gather_cp(g + NBUF, b).start()

        for b in range(NBUF):
            out_cp(nchunk - NBUF + b, b).wait()

    out = gather_kernel(table, idx)
    return out.reshape(batch, seq, d)


# speedup vs baseline: 1.0408x; 1.0408x over previous
import functools

import jax
from jax import lax
import jax.numpy as jnp
from jax.experimental import pallas as pl
from jax.experimental.pallas import tpu as pltpu
from jax.experimental.pallas import tpu_sc as plsc

NC, NS = 2, 16
NW = NC * NS
CHUNK = 32
NBUF = 4


def kernel(input_ids, table):
    batch, seq = input_ids.shape
    n = batch * seq
    _, d = table.shape
    b_per_w = n // NW
    nchunk = b_per_w // CHUNK
    idx = input_ids.reshape(n).astype(jnp.int32)

    mesh = plsc.VectorSubcoreMesh(core_axis_name="c", subcore_axis_name="s")

    @functools.partial(
        pl.kernel,
        out_type=jax.ShapeDtypeStruct((n, d), table.dtype),
        mesh=mesh,
        scratch_types=[
            pltpu.VMEM((b_per_w,), jnp.int32),
            pltpu.VMEM((NBUF, CHUNK, d), jnp.float32),
        ] + [pltpu.SemaphoreType.DMA] * (2 * NBUF),
    )
    def gather_kernel(tab_hbm, idx_hbm, out_hbm, idx_v, rows_v, *sems):
        gsems = sems[:NBUF]
        osems = sems[NBUF:]
        wid = lax.axis_index("s") * NC + lax.axis_index("c")
        base = wid * b_per_w
        pltpu.sync_copy(idx_hbm.at[pl.ds(base, b_per_w)], idx_v)

        def gather_cp(g, b):
            return pltpu.make_async_copy(
                tab_hbm.at[idx_v.at[pl.ds(g * CHUNK, CHUNK)]],
                rows_v.at[b], gsems[b])

        def out_cp(g, b):
            return pltpu.make_async_copy(
                rows_v.at[b], out_hbm.at[pl.ds(base + g * CHUNK, CHUNK)],
                osems[b])

        for b in range(NBUF):
            gather_cp(b, b).start()

        @pl.loop(0, nchunk, step=NBUF)
        def _(c):
            for b in range(NBUF):
                g = c + b
                gather_cp(g, b).wait()
                out_cp(g, b).start()
                if b >= 1:
                    gp = c + b - 1

                    @pl.when(gp + NBUF < nchunk)
                    def _():
                        out_cp(gp, b - 1).wait()
                        gather_cp(gp + NBUF, b - 1).start()
            gl = c + NBUF - 1

            @pl.when(gl + NBUF < nchunk)
            def _():
                out_cp(gl, NBUF - 1).wait()
                gather_cp(gl + NBUF, NBUF - 1).start()

        for b in range(NBUF):
            out_cp(nchunk - NBUF + b, b).wait()

    out = gather_kernel(table, idx)
    return out.reshape(batch, seq, d)


# speedup vs baseline: 1.0510x; 1.0098x over previous
import functools

import jax
from jax import lax
import jax.numpy as jnp
from jax.experimental import pallas as pl
from jax.experimental.pallas import tpu as pltpu
from jax.experimental.pallas import tpu_sc as plsc

NC, NS = 2, 16
NW = NC * NS
CHUNK = 32
NBUF = 4


def kernel(input_ids, table):
    batch, seq = input_ids.shape
    n = batch * seq
    _, d = table.shape
    b_per_w = n // NW
    nchunk = b_per_w // CHUNK
    idx = input_ids.astype(jnp.int32)
    w_per_row = seq // b_per_w

    mesh = plsc.VectorSubcoreMesh(core_axis_name="c", subcore_axis_name="s")

    @functools.partial(
        pl.kernel,
        out_type=jax.ShapeDtypeStruct((n, d), table.dtype),
        mesh=mesh,
        scratch_types=[
            pltpu.VMEM((b_per_w,), jnp.int32),
            pltpu.VMEM((NBUF, CHUNK, d), jnp.float32),
        ] + [pltpu.SemaphoreType.DMA] * (2 * NBUF),
    )
    def gather_kernel(tab_hbm, idx_hbm, out_hbm, idx_v, rows_v, *sems):
        gsems = sems[:NBUF]
        osems = sems[NBUF:]
        wid = lax.axis_index("s") * NC + lax.axis_index("c")
        base = wid * b_per_w
        pltpu.sync_copy(
            idx_hbm.at[wid // w_per_row,
                       pl.ds((wid % w_per_row) * b_per_w, b_per_w)],
            idx_v)

        def gather_cp(g, b):
            return pltpu.make_async_copy(
                tab_hbm.at[idx_v.at[pl.ds(g * CHUNK, CHUNK)]],
                rows_v.at[b], gsems[b])

        def out_cp(g, b):
            return pltpu.make_async_copy(
                rows_v.at[b], out_hbm.at[pl.ds(base + g * CHUNK, CHUNK)],
                osems[b])

        for b in range(NBUF):
            gather_cp(b, b).start()

        @pl.loop(0, nchunk, step=NBUF)
        def _(c):
            for b in range(NBUF):
                g = c + b
                gather_cp(g, b).wait()
                out_cp(g, b).start()

                @pl.when(g + NBUF < nchunk)
                def _():
                    out_cp(g, b).wait()
                    gather_cp(g + NBUF, b).start()

        for b in range(NBUF):
            out_cp(nchunk - NBUF + b, b).wait()

    out = gather_kernel(table, idx)
    return out.reshape(batch, seq, d)
